# baseline (speedup 1.0000x reference)
import functools

import jax
import jax.numpy as jnp
from jax import lax
from jax.experimental import pallas as pl
from jax.experimental.pallas import tpu as pltpu
from jax.experimental.pallas import tpu_sc as plsc

NC = 2
NS = 16
NW = NC * NS

D = 128
C_SZ = 128


def _flat_zero(buf, n_words):
    z = jnp.zeros((16,), jnp.float32)
    cols = buf.shape[-1]

    def body(i, _):
        r = i // (cols // 16)
        c = (i % (cols // 16)) * 16
        buf[r, pl.ds(c, 16)] = z
        return 0

    lax.fori_loop(0, n_words // 16, body, 0)


def _make_deg_kernel(n_pad, n_edges):
    npt = n_pad // NS
    epw = n_edges // NW
    nch = epw // C_SZ
    n_wr = 5
    wr = npt // n_wr
    mesh = plsc.VectorSubcoreMesh(core_axis_name="c", subcore_axis_name="s")

    @functools.partial(
        pl.kernel,
        out_type=jax.ShapeDtypeStruct((NC, n_pad, D), jnp.float32),
        mesh=mesh,
        scratch_types=[
            pltpu.VMEM_SHARED((n_pad, D), jnp.float32),
            [pltpu.VMEM((C_SZ,), jnp.int32)] * 2,
            pltpu.VMEM((C_SZ, D), jnp.float32),
            pltpu.VMEM((wr, D), jnp.float32),
            [pltpu.SemaphoreType.DMA] * 2,
        ],
    )
    def deg_kernel(row1, deg_out, dacc, idx, ones, zbuf, sem_i):
        cid = lax.axis_index("c")
        sid = lax.axis_index("s")
        wid = cid * NS + sid
        nb = sid * npt
        eb = wid * epw

        _flat_zero(zbuf, wr * D)
        one = jnp.ones((16,), jnp.float32)

        def fill_ones(i, _):
            r = i // (D // 16)
            c = (i % (D // 16)) * 16
            ones[r, pl.ds(c, 16)] = one
            return 0

        lax.fori_loop(0, C_SZ * D // 16, fill_ones, 0)
        for k in range(n_wr):
            pltpu.sync_copy(zbuf, dacc.at[pl.ds(nb + k * wr, wr)])
        plsc.subcore_barrier()

        def load_idx(j, b):
            pltpu.async_copy(row1.at[pl.ds(eb + j * C_SZ, C_SZ)], idx[b],
                             sem_i[b])

        def wait_idx(j, b):
            pltpu.make_async_copy(row1.at[pl.ds(eb + j * C_SZ, C_SZ)],
                                  idx[b], sem_i[b]).wait()

        load_idx(0, 0)

        def step(j, b):
            @pl.when(j + 1 < nch)
            def _():
                load_idx(j + 1, 1 - b)
            wait_idx(j, b)
            pltpu.sync_copy(ones, dacc.at[idx[b]], add=True)

        def pair(j2, _):
            j = j2 * 2
            step(j, 0)
            step(j + 1, 1)
            return 0

        lax.fori_loop(0, nch // 2, pair, 0)
        if nch % 2 == 1:
            step(nch - 1, (nch - 1) % 2)
        plsc.subcore_barrier()
        for k in range(n_wr):
            pltpu.sync_copy(dacc.at[pl.ds(nb + k * wr, wr)], zbuf)
            pltpu.sync_copy(zbuf, deg_out.at[cid, pl.ds(nb + k * wr, wr)])

    return deg_kernel


def _make_scatter_kernel(n_pad, n_edges):
    npt = n_pad // NS
    epw = n_edges // NW
    nch = epw // C_SZ
    wr = C_SZ
    n_wr = npt // wr
    mesh = plsc.VectorSubcoreMesh(core_axis_name="c", subcore_axis_name="s")

    @functools.partial(
        pl.kernel,
        out_type=jax.ShapeDtypeStruct((NC, n_pad, D), jnp.float32),
        mesh=mesh,
        scratch_types=[
            pltpu.VMEM_SHARED((n_pad, D), jnp.float32),
            [pltpu.VMEM((C_SZ,), jnp.int32)] * 2,
            [pltpu.VMEM((C_SZ,), jnp.int32)] * 2,
            [pltpu.VMEM((C_SZ, D), jnp.float32)] * 2,
            [pltpu.SemaphoreType.DMA] * 2,
            [pltpu.SemaphoreType.DMA] * 2,
        ],
    )
    def scatter_kernel(g, row1, col1, sp_out, acc, ridx, cidx, rows,
                       sem_g, sem_i):
        cid = lax.axis_index("c")
        sid = lax.axis_index("s")
        wid = cid * NS + sid
        nb = sid * npt
        eb = wid * epw
        zbuf = rows[0]

        _flat_zero(zbuf, wr * D)
        for k in range(n_wr):
            pltpu.sync_copy(zbuf, acc.at[pl.ds(nb + k * wr, wr)])
        plsc.subcore_barrier()

        def load_idx(j, b):
            pltpu.async_copy(row1.at[pl.ds(eb + j * C_SZ, C_SZ)], ridx[b],
                             sem_i[b])
            pltpu.async_copy(col1.at[pl.ds(eb + j * C_SZ, C_SZ)], cidx[b],
                             sem_i[b])

        def wait_idx(j, b):
            pltpu.make_async_copy(row1.at[pl.ds(eb + j * C_SZ, C_SZ)],
                                  ridx[b], sem_i[b]).wait()
            pltpu.make_async_copy(col1.at[pl.ds(eb + j * C_SZ, C_SZ)],
                                  cidx[b], sem_i[b]).wait()

        def wait_gather(b):
            pltpu.make_async_copy(g.at[cidx[b]], rows[b], sem_g[b]).wait()

        load_idx(0, 0)
        wait_idx(0, 0)
        pltpu.async_copy(g.at[cidx[0]], rows[0], sem_g[0])
        load_idx(1, 1)

        def step(j, b):
            nxt = 1 - b
            wait_idx(j + 1, nxt)
            pltpu.async_copy(g.at[cidx[nxt]], rows[nxt], sem_g[nxt])
            wait_gather(b)
            pltpu.sync_copy(rows[b], acc.at[ridx[b]], add=True)

            @pl.when(j + 2 < nch)
            def _():
                load_idx(j + 2, b)

        def pair(j2, _):
            j = j2 * 2
            step(j, 0)
            step(j + 1, 1)
            return 0

        lax.fori_loop(0, (nch - 1) // 2, pair, 0)
        last = nch - 1
        if last % 2 == 1:
            step(last - 1, (last - 1) % 2)
        wait_gather(last % 2)
        pltpu.sync_copy(rows[last % 2], acc.at[ridx[last % 2]], add=True)
        plsc.subcore_barrier()
        for k in range(n_wr):
            pltpu.sync_copy(acc.at[pl.ds(nb + k * wr, wr)], zbuf)
            pltpu.sync_copy(zbuf, sp_out.at[cid, pl.ds(nb + k * wr, wr)])

    return scatter_kernel


def _scale_body(deg_ref, x_ref, g_ref, dinv_ref):
    d = deg_ref[0] + deg_ref[1]
    dinv = lax.rsqrt(1.0 + d)
    g_ref[...] = dinv * x_ref[...]
    dinv_ref[...] = dinv


def _mlp_body(relu, sp_ref, g_ref, dinv_ref, w_ref, b_ref, o_ref):
    dinv = dinv_ref[...]
    h1 = dinv * (sp_ref[0] + sp_ref[1] + g_ref[...])
    h = lax.dot_general(h1, w_ref[...], (((1,), (1,)), ((), ())),
                        preferred_element_type=jnp.float32) + b_ref[...]
    if relu:
        o_ref[...] = dinv * jnp.maximum(h, 0.0)
    else:
        o_ref[...] = h


def _tc_scale(deg_parts, x, nb):
    n = x.shape[0]
    return pl.pallas_call(
        _scale_body,
        grid=(n // nb,),
        in_specs=[
            pl.BlockSpec((NC, nb, D), lambda i: (0, i, 0)),
            pl.BlockSpec((nb, D), lambda i: (i, 0)),
        ],
        out_specs=[
            pl.BlockSpec((nb, D), lambda i: (i, 0)),
            pl.BlockSpec((nb, D), lambda i: (i, 0)),
        ],
        out_shape=[
            jax.ShapeDtypeStruct((n, D), jnp.float32),
            jax.ShapeDtypeStruct((n, D), jnp.float32),
        ],
    )(deg_parts, x)


def _tc_mlp(sp, g, dinvb, w, b2d, nb, relu):
    n = g.shape[0]
    return pl.pallas_call(
        functools.partial(_mlp_body, relu),
        grid=(n // nb,),
        in_specs=[
            pl.BlockSpec((NC, nb, D), lambda i: (0, i, 0)),
            pl.BlockSpec((nb, D), lambda i: (i, 0)),
            pl.BlockSpec((nb, D), lambda i: (i, 0)),
            pl.BlockSpec((D, D), lambda i: (0, 0)),
            pl.BlockSpec((1, D), lambda i: (0, 0)),
        ],
        out_specs=pl.BlockSpec((nb, D), lambda i: (i, 0)),
        out_shape=jax.ShapeDtypeStruct((n, D), jnp.float32),
    )(sp, g, dinvb, w, b2d)


@jax.jit
def kernel(x, edge_index, W1, b1, W2, b2):
    n = x.shape[0]
    e = edge_index.shape[1]
    n_pad = ((n + NW * 8 - 1) // (NW * 8)) * (NW * 8)
    nb = 1000

    e_pad = ((e + NW * C_SZ - 1) // (NW * C_SZ)) * (NW * C_SZ)
    pad = e_pad - e
    row1 = jnp.concatenate(
        [edge_index[0].astype(jnp.int32),
         jnp.full((pad,), n_pad - 1, jnp.int32)])
    col1 = jnp.concatenate(
        [edge_index[1].astype(jnp.int32), jnp.zeros((pad,), jnp.int32)])

    deg_parts = _make_deg_kernel(n_pad, e_pad)(row1)
    g, dinvb = _tc_scale(deg_parts, x, nb)

    edge_scatter = _make_scatter_kernel(n_pad, e_pad)
    s1 = edge_scatter(g, row1, col1)
    g2 = _tc_mlp(s1, g, dinvb, W1, b1.reshape(1, D), nb, relu=True)
    s2 = edge_scatter(g2, row1, col1)
    return _tc_mlp(s2, g2, dinvb, W2, b2.reshape(1, D), nb, relu=False)


# speedup vs baseline: 1.0322x; 1.0322x over previous
import functools

import jax
import jax.numpy as jnp
from jax import lax
from jax.experimental import pallas as pl
from jax.experimental.pallas import tpu as pltpu
from jax.experimental.pallas import tpu_sc as plsc

NC = 2
NS = 16
NW = NC * NS

D = 128
C_SZ = 128


def _flat_zero(buf, n_words):
    z = jnp.zeros((16,), jnp.float32)
    cols = buf.shape[-1]

    def body(i, _):
        r = i // (cols // 16)
        c = (i % (cols // 16)) * 16
        buf[r, pl.ds(c, 16)] = z
        return 0

    lax.fori_loop(0, n_words // 16, body, 0)


def _make_deg_kernel(n_pad, n_edges):
    npt = n_pad // NS
    epw = n_edges // NW
    nch = epw // C_SZ
    n_wr = 5
    wr = npt // n_wr
    mesh = plsc.VectorSubcoreMesh(core_axis_name="c", subcore_axis_name="s")

    @functools.partial(
        pl.kernel,
        out_type=jax.ShapeDtypeStruct((NC, n_pad, D), jnp.float32),
        mesh=mesh,
        scratch_types=[
            pltpu.VMEM_SHARED((n_pad, D), jnp.float32),
            [pltpu.VMEM((C_SZ,), jnp.int32)] * 2,
            pltpu.VMEM((C_SZ, D), jnp.float32),
            pltpu.VMEM((wr, D), jnp.float32),
            [pltpu.SemaphoreType.DMA] * 2,
        ],
    )
    def deg_kernel(row1, deg_out, dacc, idx, ones, zbuf, sem_i):
        cid = lax.axis_index("c")
        sid = lax.axis_index("s")
        wid = cid * NS + sid
        nb = sid * npt
        eb = wid * epw

        _flat_zero(zbuf, wr * D)
        one = jnp.ones((16,), jnp.float32)

        def fill_ones(i, _):
            r = i // (D // 16)
            c = (i % (D // 16)) * 16
            ones[r, pl.ds(c, 16)] = one
            return 0

        lax.fori_loop(0, C_SZ * D // 16, fill_ones, 0)
        for k in range(n_wr):
            pltpu.sync_copy(zbuf, dacc.at[pl.ds(nb + k * wr, wr)])
        plsc.subcore_barrier()

        def load_idx(j, b):
            pltpu.async_copy(row1.at[pl.ds(eb + j * C_SZ, C_SZ)], idx[b],
                             sem_i[b])

        def wait_idx(j, b):
            pltpu.make_async_copy(row1.at[pl.ds(eb + j * C_SZ, C_SZ)],
                                  idx[b], sem_i[b]).wait()

        load_idx(0, 0)

        def step(j, b):
            @pl.when(j + 1 < nch)
            def _():
                load_idx(j + 1, 1 - b)
            wait_idx(j, b)
            pltpu.sync_copy(ones, dacc.at[idx[b]], add=True)

        def pair(j2, _):
            j = j2 * 2
            step(j, 0)
            step(j + 1, 1)
            return 0

        lax.fori_loop(0, nch // 2, pair, 0)
        if nch % 2 == 1:
            step(nch - 1, (nch - 1) % 2)
        plsc.subcore_barrier()
        for k in range(n_wr):
            pltpu.sync_copy(dacc.at[pl.ds(nb + k * wr, wr)], zbuf)
            pltpu.sync_copy(zbuf, deg_out.at[cid, pl.ds(nb + k * wr, wr)])

    return deg_kernel


def _make_scatter_kernel(n_pad, n_edges):
    npt = n_pad // NS
    epw = n_edges // NW
    nch = epw // C_SZ
    wr = C_SZ
    n_wr = npt // wr
    mesh = plsc.VectorSubcoreMesh(core_axis_name="c", subcore_axis_name="s")

    @functools.partial(
        pl.kernel,
        out_type=jax.ShapeDtypeStruct((NC, n_pad, D), jnp.float32),
        mesh=mesh,
        scratch_types=[
            pltpu.VMEM_SHARED((n_pad, D), jnp.float32),
            [pltpu.VMEM((C_SZ,), jnp.int32)] * 2,
            [pltpu.VMEM((C_SZ,), jnp.int32)] * 2,
            [pltpu.VMEM((C_SZ, D), jnp.float32)] * 2,
            [pltpu.SemaphoreType.DMA] * 2,
            [pltpu.SemaphoreType.DMA] * 2,
        ],
    )
    def scatter_kernel(g, row1, col1, sp_out, acc, ridx, cidx, rows,
                       sem_g, sem_i):
        cid = lax.axis_index("c")
        sid = lax.axis_index("s")
        wid = cid * NS + sid
        nb = sid * npt
        eb = wid * epw
        zbuf = rows[0]

        _flat_zero(zbuf, wr * D)
        for k in range(n_wr):
            pltpu.sync_copy(zbuf, acc.at[pl.ds(nb + k * wr, wr)])
        plsc.subcore_barrier()

        def load_idx(j, b):
            pltpu.async_copy(row1.at[pl.ds(eb + j * C_SZ, C_SZ)], ridx[b],
                             sem_i[b])
            pltpu.async_copy(col1.at[pl.ds(eb + j * C_SZ, C_SZ)], cidx[b],
                             sem_i[b])

        def wait_idx(j, b):
            pltpu.make_async_copy(row1.at[pl.ds(eb + j * C_SZ, C_SZ)],
                                  ridx[b], sem_i[b]).wait()
            pltpu.make_async_copy(col1.at[pl.ds(eb + j * C_SZ, C_SZ)],
                                  cidx[b], sem_i[b]).wait()

        def wait_gather(b):
            pltpu.make_async_copy(g.at[cidx[b]], rows[b], sem_g[b]).wait()

        load_idx(0, 0)
        wait_idx(0, 0)
        pltpu.async_copy(g.at[cidx[0]], rows[0], sem_g[0])
        load_idx(1, 1)

        def step(j, b):
            nxt = 1 - b
            wait_idx(j + 1, nxt)
            pltpu.async_copy(g.at[cidx[nxt]], rows[nxt], sem_g[nxt])
            wait_gather(b)
            pltpu.sync_copy(rows[b], acc.at[ridx[b]], add=True)

            @pl.when(j + 2 < nch)
            def _():
                load_idx(j + 2, b)

        def pair(j2, _):
            j = j2 * 2
            step(j, 0)
            step(j + 1, 1)
            return 0

        lax.fori_loop(0, (nch - 1) // 2, pair, 0)
        last = nch - 1
        if last % 2 == 1:
            step(last - 1, (last - 1) % 2)
        wait_gather(last % 2)
        pltpu.sync_copy(rows[last % 2], acc.at[ridx[last % 2]], add=True)
        plsc.subcore_barrier()
        for k in range(n_wr):
            pltpu.sync_copy(acc.at[pl.ds(nb + k * wr, wr)], zbuf)
            pltpu.sync_copy(zbuf, sp_out.at[cid, pl.ds(nb + k * wr, wr)])

    return scatter_kernel


def _scale_body(deg_ref, x_ref, g_ref, dinv_ref):
    d = deg_ref[0] + deg_ref[1]
    dinv = lax.rsqrt(1.0 + d)
    g_ref[...] = dinv * x_ref[...]
    dinv_ref[...] = dinv


def _mlp_body(relu, sp_ref, g_ref, dinv_ref, w_ref, b_ref, o_ref):
    dinv = dinv_ref[...]
    h1 = dinv * (sp_ref[0] + sp_ref[1] + g_ref[...])
    h = lax.dot_general(h1, w_ref[...], (((1,), (1,)), ((), ())),
                        preferred_element_type=jnp.float32) + b_ref[...]
    if relu:
        o_ref[...] = dinv * jnp.maximum(h, 0.0)
    else:
        o_ref[...] = h


def _tc_scale(deg_parts, x, nb):
    n = x.shape[0]
    return pl.pallas_call(
        _scale_body,
        grid=(n // nb,),
        in_specs=[
            pl.BlockSpec((NC, nb, D), lambda i: (0, i, 0)),
            pl.BlockSpec((nb, D), lambda i: (i, 0)),
        ],
        out_specs=[
            pl.BlockSpec((nb, D), lambda i: (i, 0)),
            pl.BlockSpec((nb, D), lambda i: (i, 0)),
        ],
        out_shape=[
            jax.ShapeDtypeStruct((n, D), jnp.float32),
            jax.ShapeDtypeStruct((n, D), jnp.float32),
        ],
    )(deg_parts, x)


def _tc_mlp(sp, g, dinvb, w, b2d, nb, relu):
    n = g.shape[0]
    return pl.pallas_call(
        functools.partial(_mlp_body, relu),
        grid=(n // nb,),
        in_specs=[
            pl.BlockSpec((NC, nb, D), lambda i: (0, i, 0)),
            pl.BlockSpec((nb, D), lambda i: (i, 0)),
            pl.BlockSpec((nb, D), lambda i: (i, 0)),
            pl.BlockSpec((D, D), lambda i: (0, 0)),
            pl.BlockSpec((1, D), lambda i: (0, 0)),
        ],
        out_specs=pl.BlockSpec((nb, D), lambda i: (i, 0)),
        out_shape=jax.ShapeDtypeStruct((n, D), jnp.float32),
    )(sp, g, dinvb, w, b2d)


@jax.jit
def kernel(x, edge_index, W1, b1, W2, b2):
    n = x.shape[0]
    e = edge_index.shape[1]
    n_pad = ((n + NW * 8 - 1) // (NW * 8)) * (NW * 8)
    nb = 1000

    e_pad = ((e + NW * C_SZ - 1) // (NW * C_SZ)) * (NW * C_SZ)
    pad = e_pad - e
    row1 = jnp.concatenate(
        [edge_index[0].astype(jnp.int32),
         n + jnp.arange(pad, dtype=jnp.int32) % (n_pad - n)])
    col1 = jnp.concatenate(
        [edge_index[1].astype(jnp.int32), jnp.zeros((pad,), jnp.int32)])

    deg_parts = _make_deg_kernel(n_pad, e_pad)(row1)
    g, dinvb = _tc_scale(deg_parts, x, nb)

    edge_scatter = _make_scatter_kernel(n_pad, e_pad)
    s1 = edge_scatter(g, row1, col1)
    g2 = _tc_mlp(s1, g, dinvb, W1, b1.reshape(1, D), nb, relu=True)
    s2 = edge_scatter(g2, row1, col1)
    return _tc_mlp(s2, g2, dinvb, W2, b2.reshape(1, D), nb, relu=False)


# speedup vs baseline: 1.8272x; 1.7702x over previous
import functools

import jax
import jax.numpy as jnp
from jax import lax
from jax.experimental import pallas as pl
from jax.experimental.pallas import tpu as pltpu
from jax.experimental.pallas import tpu_sc as plsc

NC = 2
NS = 16
NW = NC * NS

D = 128
C_SZ = 128


def _flat_zero(buf, n_words):
    z = jnp.zeros((16,), jnp.float32)
    cols = buf.shape[-1]

    def body(i, _):
        r = i // (cols // 16)
        c = (i % (cols // 16)) * 16
        buf[r, pl.ds(c, 16)] = z
        return 0

    lax.fori_loop(0, n_words // 16, body, 0)


def _make_deg_kernel(n_pad, n_edges):
    npt = n_pad // NS
    epw = n_edges // NW
    nch = epw // C_SZ
    n_wr = 5
    wr = npt // n_wr
    mesh = plsc.VectorSubcoreMesh(core_axis_name="c", subcore_axis_name="s")

    @functools.partial(
        pl.kernel,
        out_type=jax.ShapeDtypeStruct((NC, n_pad, D), jnp.float32),
        mesh=mesh,
        scratch_types=[
            pltpu.VMEM_SHARED((n_pad, D), jnp.float32),
            [pltpu.VMEM((C_SZ,), jnp.int32)] * 2,
            pltpu.VMEM((C_SZ, D), jnp.float32),
            pltpu.VMEM((wr, D), jnp.float32),
            [pltpu.SemaphoreType.DMA] * 2,
        ],
    )
    def deg_kernel(row1, deg_out, dacc, idx, ones, zbuf, sem_i):
        cid = lax.axis_index("c")
        sid = lax.axis_index("s")
        wid = cid * NS + sid
        nb = sid * npt
        eb = wid * epw

        _flat_zero(zbuf, wr * D)
        one = jnp.ones((16,), jnp.float32)

        def fill_ones(i, _):
            r = i // (D // 16)
            c = (i % (D // 16)) * 16
            ones[r, pl.ds(c, 16)] = one
            return 0

        lax.fori_loop(0, C_SZ * D // 16, fill_ones, 0)
        for k in range(n_wr):
            pltpu.sync_copy(zbuf, dacc.at[pl.ds(nb + k * wr, wr)])
        plsc.subcore_barrier()

        def load_idx(j, b):
            pltpu.async_copy(row1.at[pl.ds(eb + j * C_SZ, C_SZ)], idx[b],
                             sem_i[b])

        def wait_idx(j, b):
            pltpu.make_async_copy(row1.at[pl.ds(eb + j * C_SZ, C_SZ)],
                                  idx[b], sem_i[b]).wait()

        load_idx(0, 0)

        def step(j, b):
            @pl.when(j + 1 < nch)
            def _():
                load_idx(j + 1, 1 - b)
            wait_idx(j, b)
            pltpu.sync_copy(ones, dacc.at[idx[b]], add=True)

        def pair(j2, _):
            j = j2 * 2
            step(j, 0)
            step(j + 1, 1)
            return 0

        lax.fori_loop(0, nch // 2, pair, 0)
        if nch % 2 == 1:
            step(nch - 1, (nch - 1) % 2)
        plsc.subcore_barrier()
        for k in range(n_wr):
            pltpu.sync_copy(dacc.at[pl.ds(nb + k * wr, wr)], zbuf)
            pltpu.sync_copy(zbuf, deg_out.at[cid, pl.ds(nb + k * wr, wr)])

    return deg_kernel


def _make_scatter_kernel(n_pad, n_edges):
    npt = n_pad // NS
    epw = n_edges // NW
    nch = epw // C_SZ
    wr = C_SZ
    n_wr = npt // wr
    mesh = plsc.VectorSubcoreMesh(core_axis_name="c", subcore_axis_name="s")

    @functools.partial(
        pl.kernel,
        out_type=jax.ShapeDtypeStruct((NC, n_pad, D), jnp.float32),
        mesh=mesh,
        scratch_types=[
            pltpu.VMEM_SHARED((n_pad, D), jnp.float32),
            [pltpu.VMEM((C_SZ,), jnp.int32)] * 2,
            [pltpu.VMEM((C_SZ,), jnp.int32)] * 2,
            [pltpu.VMEM((C_SZ, D), jnp.float32)] * 2,
            [pltpu.SemaphoreType.DMA] * 2,
            [pltpu.SemaphoreType.DMA] * 2,
        ],
    )
    def scatter_kernel(g, row1, col1, sp_out, acc, ridx, cidx, rows,
                       sem_g, sem_i):
        cid = lax.axis_index("c")
        sid = lax.axis_index("s")
        wid = cid * NS + sid
        nb = sid * npt
        eb = wid * epw
        zbuf = rows[0]

        _flat_zero(zbuf, wr * D)
        for k in range(n_wr):
            pltpu.sync_copy(zbuf, acc.at[pl.ds(nb + k * wr, wr)])
        plsc.subcore_barrier()

        def load_idx(j, b):
            pltpu.async_copy(row1.at[pl.ds(eb + j * C_SZ, C_SZ)], ridx[b],
                             sem_i[b])
            pltpu.async_copy(col1.at[pl.ds(eb + j * C_SZ, C_SZ)], cidx[b],
                             sem_i[b])

        def wait_idx(j, b):
            pltpu.make_async_copy(row1.at[pl.ds(eb + j * C_SZ, C_SZ)],
                                  ridx[b], sem_i[b]).wait()
            pltpu.make_async_copy(col1.at[pl.ds(eb + j * C_SZ, C_SZ)],
                                  cidx[b], sem_i[b]).wait()

        def wait_gather(b):
            pltpu.make_async_copy(g.at[cidx[b]], rows[b], sem_g[b]).wait()

        load_idx(0, 0)
        wait_idx(0, 0)
        pltpu.async_copy(g.at[cidx[0]], rows[0], sem_g[0])
        load_idx(1, 1)

        def step(j, b):
            nxt = 1 - b
            wait_idx(j + 1, nxt)
            pltpu.async_copy(g.at[cidx[nxt]], rows[nxt], sem_g[nxt])
            wait_gather(b)
            pltpu.sync_copy(rows[b], acc.at[ridx[b]], add=True)

            @pl.when(j + 2 < nch)
            def _():
                load_idx(j + 2, b)

        def pair(j2, _):
            j = j2 * 2
            step(j, 0)
            step(j + 1, 1)
            return 0

        lax.fori_loop(0, (nch - 1) // 2, pair, 0)
        last = nch - 1
        if last % 2 == 1:
            step(last - 1, (last - 1) % 2)
        wait_gather(last % 2)
        pltpu.sync_copy(rows[last % 2], acc.at[ridx[last % 2]], add=True)
        plsc.subcore_barrier()
        for k in range(n_wr):
            pltpu.sync_copy(acc.at[pl.ds(nb + k * wr, wr)], zbuf)
            pltpu.sync_copy(zbuf, sp_out.at[cid, pl.ds(nb + k * wr, wr)])

    return scatter_kernel


def _scale_body(deg_ref, x_ref, g_ref, dinv_ref):
    d = deg_ref[0] + deg_ref[1]
    dinv = lax.rsqrt(1.0 + d)
    g_ref[...] = dinv * x_ref[...]
    dinv_ref[...] = dinv


def _mlp_body(relu, sp_ref, g_ref, dinv_ref, w_ref, b_ref, o_ref):
    dinv = dinv_ref[...]
    h1 = dinv * (sp_ref[0] + sp_ref[1] + g_ref[...])
    h = lax.dot_general(h1, w_ref[...], (((1,), (1,)), ((), ())),
                        preferred_element_type=jnp.float32) + b_ref[...]
    if relu:
        o_ref[...] = dinv * jnp.maximum(h, 0.0)
    else:
        o_ref[...] = h


def _tc_scale(deg_parts, x, nb):
    n = x.shape[0]
    return pl.pallas_call(
        _scale_body,
        grid=(n // nb,),
        in_specs=[
            pl.BlockSpec((NC, nb, D), lambda i: (0, i, 0)),
            pl.BlockSpec((nb, D), lambda i: (i, 0)),
        ],
        out_specs=[
            pl.BlockSpec((nb, D), lambda i: (i, 0)),
            pl.BlockSpec((nb, D), lambda i: (i, 0)),
        ],
        out_shape=[
            jax.ShapeDtypeStruct((n, D), jnp.float32),
            jax.ShapeDtypeStruct((n, D), jnp.float32),
        ],
    )(deg_parts, x)


def _tc_mlp(sp, g, dinvb, w, b2d, nb, relu):
    n = g.shape[0]
    return pl.pallas_call(
        functools.partial(_mlp_body, relu),
        grid=(n // nb,),
        in_specs=[
            pl.BlockSpec((NC, nb, D), lambda i: (0, i, 0)),
            pl.BlockSpec((nb, D), lambda i: (i, 0)),
            pl.BlockSpec((nb, D), lambda i: (i, 0)),
            pl.BlockSpec((D, D), lambda i: (0, 0)),
            pl.BlockSpec((1, D), lambda i: (0, 0)),
        ],
        out_specs=pl.BlockSpec((nb, D), lambda i: (i, 0)),
        out_shape=jax.ShapeDtypeStruct((n, D), jnp.float32),
    )(sp, g, dinvb, w, b2d)


@jax.jit
def kernel(x, edge_index, W1, b1, W2, b2):
    n = x.shape[0]
    e = edge_index.shape[1]
    n_pad = ((n + NW * 8 - 1) // (NW * 8)) * (NW * 8)
    nb = 1000

    e_pad = ((e + NW * C_SZ - 1) // (NW * C_SZ)) * (NW * C_SZ)
    pad = e_pad - e
    row1 = jnp.concatenate(
        [edge_index[0].astype(jnp.int32),
         n + jnp.arange(pad, dtype=jnp.int32) % (n_pad - n)])
    col1 = jnp.concatenate(
        [edge_index[1].astype(jnp.int32),
         jnp.arange(pad, dtype=jnp.int32) % n])

    deg_parts = _make_deg_kernel(n_pad, e_pad)(row1)
    g, dinvb = _tc_scale(deg_parts, x, nb)

    edge_scatter = _make_scatter_kernel(n_pad, e_pad)
    s1 = edge_scatter(g, row1, col1)
    g2 = _tc_mlp(s1, g, dinvb, W1, b1.reshape(1, D), nb, relu=True)
    s2 = edge_scatter(g2, row1, col1)
    return _tc_mlp(s2, g2, dinvb, W2, b2.reshape(1, D), nb, relu=False)


# speedup vs baseline: 1.9795x; 1.0833x over previous
import functools

import jax
import jax.numpy as jnp
from jax import lax
from jax.experimental import pallas as pl
from jax.experimental.pallas import tpu as pltpu
from jax.experimental.pallas import tpu_sc as plsc

NC = 2
NS = 16
NW = NC * NS

D = 128
C_SZ = 128


def _flat_zero(buf, n_words):
    z = jnp.zeros((16,), jnp.float32)
    cols = buf.shape[-1]

    def body(i, _):
        r = i // (cols // 16)
        c = (i % (cols // 16)) * 16
        buf[r, pl.ds(c, 16)] = z
        return 0

    lax.fori_loop(0, n_words // 16, body, 0)


def _make_deg_kernel(n_pad, n_edges):
    npt = n_pad // NS
    epw = n_edges // NW
    nch = epw // C_SZ
    n_wr = 5
    wr = npt // n_wr
    mesh = plsc.VectorSubcoreMesh(core_axis_name="c", subcore_axis_name="s")

    @functools.partial(
        pl.kernel,
        out_type=jax.ShapeDtypeStruct((NC, n_pad, D), jnp.float32),
        mesh=mesh,
        scratch_types=[
            pltpu.VMEM_SHARED((n_pad, D), jnp.float32),
            [pltpu.VMEM((C_SZ,), jnp.int32)] * 2,
            pltpu.VMEM((C_SZ, D), jnp.float32),
            pltpu.VMEM((wr, D), jnp.float32),
            [pltpu.SemaphoreType.DMA] * 2,
        ],
    )
    def deg_kernel(row1, deg_out, dacc, idx, ones, zbuf, sem_i):
        cid = lax.axis_index("c")
        sid = lax.axis_index("s")
        wid = cid * NS + sid
        nb = sid * npt
        eb = wid * epw

        _flat_zero(zbuf, wr * D)
        one = jnp.ones((16,), jnp.float32)

        def fill_ones(i, _):
            r = i // (D // 16)
            c = (i % (D // 16)) * 16
            ones[r, pl.ds(c, 16)] = one
            return 0

        lax.fori_loop(0, C_SZ * D // 16, fill_ones, 0)
        for k in range(n_wr):
            pltpu.sync_copy(zbuf, dacc.at[pl.ds(nb + k * wr, wr)])
        plsc.subcore_barrier()

        def load_idx(j, b):
            pltpu.async_copy(row1.at[pl.ds(eb + j * C_SZ, C_SZ)], idx[b],
                             sem_i[b])

        def wait_idx(j, b):
            pltpu.make_async_copy(row1.at[pl.ds(eb + j * C_SZ, C_SZ)],
                                  idx[b], sem_i[b]).wait()

        load_idx(0, 0)

        def step(j, b):
            @pl.when(j + 1 < nch)
            def _():
                load_idx(j + 1, 1 - b)
            wait_idx(j, b)
            pltpu.sync_copy(ones, dacc.at[idx[b]], add=True)

        def pair(j2, _):
            j = j2 * 2
            step(j, 0)
            step(j + 1, 1)
            return 0

        lax.fori_loop(0, nch // 2, pair, 0)
        if nch % 2 == 1:
            step(nch - 1, (nch - 1) % 2)
        plsc.subcore_barrier()
        for k in range(n_wr):
            pltpu.sync_copy(dacc.at[pl.ds(nb + k * wr, wr)], zbuf)
            pltpu.sync_copy(zbuf, deg_out.at[cid, pl.ds(nb + k * wr, wr)])

    return deg_kernel


def _make_scatter_kernel(n_pad, n_edges):
    npt = n_pad // NS
    epw = n_edges // NW
    nch = epw // C_SZ
    wr = C_SZ
    n_wr = npt // wr
    mesh = plsc.VectorSubcoreMesh(core_axis_name="c", subcore_axis_name="s")

    @functools.partial(
        pl.kernel,
        out_type=jax.ShapeDtypeStruct((NC, n_pad, D), jnp.float32),
        mesh=mesh,
        scratch_types=[
            pltpu.VMEM_SHARED((n_pad, D), jnp.float32),
            [pltpu.VMEM((C_SZ,), jnp.int32)] * 3,
            [pltpu.VMEM((C_SZ,), jnp.int32)] * 3,
            [pltpu.VMEM((C_SZ, D), jnp.float32)] * 2,
            [pltpu.SemaphoreType.DMA] * 2,
            [pltpu.SemaphoreType.DMA] * 2,
            [pltpu.SemaphoreType.DMA] * 3,
        ],
    )
    def scatter_kernel(g, row1, col1, sp_out, acc, ridx, cidx, rows,
                       sem_g, sem_s, sem_i):
        cid = lax.axis_index("c")
        sid = lax.axis_index("s")
        wid = cid * NS + sid
        nb = sid * npt
        eb = wid * epw
        zbuf = rows[0]

        _flat_zero(zbuf, wr * D)
        for k in range(n_wr):
            pltpu.sync_copy(zbuf, acc.at[pl.ds(nb + k * wr, wr)])
        plsc.subcore_barrier()

        def load_idx(j, b3):
            pltpu.async_copy(row1.at[pl.ds(eb + j * C_SZ, C_SZ)], ridx[b3],
                             sem_i[b3])
            pltpu.async_copy(col1.at[pl.ds(eb + j * C_SZ, C_SZ)], cidx[b3],
                             sem_i[b3])

        def wait_idx(j, b3):
            pltpu.make_async_copy(row1.at[pl.ds(eb + j * C_SZ, C_SZ)],
                                  ridx[b3], sem_i[b3]).wait()
            pltpu.make_async_copy(col1.at[pl.ds(eb + j * C_SZ, C_SZ)],
                                  cidx[b3], sem_i[b3]).wait()

        def wait_gather(b2, b3):
            pltpu.make_async_copy(g.at[cidx[b3]], rows[b2], sem_g[b2]).wait()

        def wait_scatter(b2, b3):
            pltpu.make_async_copy(rows[b2], acc.at[ridx[b3]],
                                  sem_s[b2]).wait()

        load_idx(0, 0)
        wait_idx(0, 0)
        pltpu.async_copy(g.at[cidx[0]], rows[0], sem_g[0])
        load_idx(1, 1)

        def step(j, jb):
            b2, b3 = jb % 2, jb % 3
            n2, n3 = (jb + 1) % 2, (jb + 1) % 3

            @pl.when(j + 1 < nch)
            def _():
                wait_idx(j + 1, n3)

                @pl.when(j >= 1)
                def _():
                    wait_scatter(n2, (jb + 2) % 3)
                pltpu.async_copy(g.at[cidx[n3]], rows[n2], sem_g[n2])

            wait_gather(b2, b3)
            pltpu.async_copy(rows[b2], acc.at[ridx[b3]], sem_s[b2], add=True)

            @pl.when(j + 2 < nch)
            def _():
                load_idx(j + 2, (jb + 2) % 3)

        def six(i6, _):
            for jb in range(6):
                step(i6 * 6 + jb, jb)
            return 0

        lax.fori_loop(0, nch // 6, six, 0)
        for j in range(nch - nch % 6, nch):
            step(j, j % 6)
        last = nch - 1
        wait_scatter((last - 1) % 2, (last - 1) % 3)
        wait_scatter(last % 2, last % 3)
        plsc.subcore_barrier()
        for k in range(n_wr):
            pltpu.sync_copy(acc.at[pl.ds(nb + k * wr, wr)], zbuf)
            pltpu.sync_copy(zbuf, sp_out.at[cid, pl.ds(nb + k * wr, wr)])

    return scatter_kernel


def _scale_body(deg_ref, x_ref, g_ref, dinv_ref):
    d = deg_ref[0] + deg_ref[1]
    dinv = lax.rsqrt(1.0 + d)
    g_ref[...] = dinv * x_ref[...]
    dinv_ref[...] = dinv


def _mlp_body(relu, sp_ref, g_ref, dinv_ref, w_ref, b_ref, o_ref):
    dinv = dinv_ref[...]
    h1 = dinv * (sp_ref[0] + sp_ref[1] + g_ref[...])
    h = lax.dot_general(h1, w_ref[...], (((1,), (1,)), ((), ())),
                        preferred_element_type=jnp.float32) + b_ref[...]
    if relu:
        o_ref[...] = dinv * jnp.maximum(h, 0.0)
    else:
        o_ref[...] = h


def _tc_scale(deg_parts, x, nb):
    n = x.shape[0]
    return pl.pallas_call(
        _scale_body,
        grid=(n // nb,),
        in_specs=[
            pl.BlockSpec((NC, nb, D), lambda i: (0, i, 0)),
            pl.BlockSpec((nb, D), lambda i: (i, 0)),
        ],
        out_specs=[
            pl.BlockSpec((nb, D), lambda i: (i, 0)),
            pl.BlockSpec((nb, D), lambda i: (i, 0)),
        ],
        out_shape=[
            jax.ShapeDtypeStruct((n, D), jnp.float32),
            jax.ShapeDtypeStruct((n, D), jnp.float32),
        ],
    )(deg_parts, x)


def _tc_mlp(sp, g, dinvb, w, b2d, nb, relu):
    n = g.shape[0]
    return pl.pallas_call(
        functools.partial(_mlp_body, relu),
        grid=(n // nb,),
        in_specs=[
            pl.BlockSpec((NC, nb, D), lambda i: (0, i, 0)),
            pl.BlockSpec((nb, D), lambda i: (i, 0)),
            pl.BlockSpec((nb, D), lambda i: (i, 0)),
            pl.BlockSpec((D, D), lambda i: (0, 0)),
            pl.BlockSpec((1, D), lambda i: (0, 0)),
        ],
        out_specs=pl.BlockSpec((nb, D), lambda i: (i, 0)),
        out_shape=jax.ShapeDtypeStruct((n, D), jnp.float32),
    )(sp, g, dinvb, w, b2d)


@jax.jit
def kernel(x, edge_index, W1, b1, W2, b2):
    n = x.shape[0]
    e = edge_index.shape[1]
    n_pad = ((n + NW * 8 - 1) // (NW * 8)) * (NW * 8)
    nb = 1000

    e_pad = ((e + NW * C_SZ - 1) // (NW * C_SZ)) * (NW * C_SZ)
    pad = e_pad - e
    row1 = jnp.concatenate(
        [edge_index[0].astype(jnp.int32),
         n + jnp.arange(pad, dtype=jnp.int32) % (n_pad - n)])
    col1 = jnp.concatenate(
        [edge_index[1].astype(jnp.int32),
         jnp.arange(pad, dtype=jnp.int32) % n])

    deg_parts = _make_deg_kernel(n_pad, e_pad)(row1)
    g, dinvb = _tc_scale(deg_parts, x, nb)

    edge_scatter = _make_scatter_kernel(n_pad, e_pad)
    s1 = edge_scatter(g, row1, col1)
    g2 = _tc_mlp(s1, g, dinvb, W1, b1.reshape(1, D), nb, relu=True)
    s2 = edge_scatter(g2, row1, col1)
    return _tc_mlp(s2, g2, dinvb, W2, b2.reshape(1, D), nb, relu=False)


# speedup vs baseline: 2.0042x; 1.0125x over previous
import functools

import jax
import jax.numpy as jnp
from jax import lax
from jax.experimental import pallas as pl
from jax.experimental.pallas import tpu as pltpu
from jax.experimental.pallas import tpu_sc as plsc

NC = 2
NS = 16
NW = NC * NS

D = 128
C_SZ = 128


def _flat_zero(buf, n_words):
    z = jnp.zeros((16,), jnp.float32)
    cols = buf.shape[-1]

    def body(i, _):
        r = i // (cols // 16)
        c = (i % (cols // 16)) * 16
        buf[r, pl.ds(c, 16)] = z
        return 0

    lax.fori_loop(0, n_words // 16, body, 0)


def _make_deg_kernel(n_pad, n_edges):
    npt = n_pad // NS
    epw = n_edges // NW
    nch = epw // C_SZ
    n_wr = 5
    wr = npt // n_wr
    mesh = plsc.VectorSubcoreMesh(core_axis_name="c", subcore_axis_name="s")

    @functools.partial(
        pl.kernel,
        out_type=jax.ShapeDtypeStruct((NC, n_pad, D), jnp.float32),
        mesh=mesh,
        scratch_types=[
            pltpu.VMEM_SHARED((n_pad, D), jnp.float32),
            [pltpu.VMEM((C_SZ,), jnp.int32)] * 2,
            pltpu.VMEM((C_SZ, D), jnp.float32),
            pltpu.VMEM((wr, D), jnp.float32),
            [pltpu.SemaphoreType.DMA] * 2,
        ],
    )
    def deg_kernel(row1, deg_out, dacc, idx, ones, zbuf, sem_i):
        cid = lax.axis_index("c")
        sid = lax.axis_index("s")
        wid = cid * NS + sid
        nb = sid * npt
        eb = wid * epw

        _flat_zero(zbuf, wr * D)
        one = jnp.ones((16,), jnp.float32)

        def fill_ones(i, _):
            r = i // (D // 16)
            c = (i % (D // 16)) * 16
            ones[r, pl.ds(c, 16)] = one
            return 0

        lax.fori_loop(0, C_SZ * D // 16, fill_ones, 0)
        for k in range(n_wr):
            pltpu.sync_copy(zbuf, dacc.at[pl.ds(nb + k * wr, wr)])
        plsc.subcore_barrier()

        def load_idx(j, b):
            pltpu.async_copy(row1.at[pl.ds(eb + j * C_SZ, C_SZ)], idx[b],
                             sem_i[b])

        def wait_idx(j, b):
            pltpu.make_async_copy(row1.at[pl.ds(eb + j * C_SZ, C_SZ)],
                                  idx[b], sem_i[b]).wait()

        load_idx(0, 0)

        def step(j, b):
            @pl.when(j + 1 < nch)
            def _():
                load_idx(j + 1, 1 - b)
            wait_idx(j, b)
            pltpu.sync_copy(ones, dacc.at[idx[b]], add=True)

        def pair(j2, _):
            j = j2 * 2
            step(j, 0)
            step(j + 1, 1)
            return 0

        lax.fori_loop(0, nch // 2, pair, 0)
        if nch % 2 == 1:
            step(nch - 1, (nch - 1) % 2)
        plsc.subcore_barrier()
        for k in range(n_wr):
            pltpu.sync_copy(dacc.at[pl.ds(nb + k * wr, wr)], zbuf)
            pltpu.sync_copy(zbuf, deg_out.at[cid, pl.ds(nb + k * wr, wr)])

    return deg_kernel


def _make_scatter_kernel(n_pad, n_edges):
    npt = n_pad // NS
    epw = n_edges // NW
    nch = epw // C_SZ
    wr = C_SZ
    n_wr = npt // wr
    mesh = plsc.VectorSubcoreMesh(core_axis_name="c", subcore_axis_name="s")

    @functools.partial(
        pl.kernel,
        out_type=jax.ShapeDtypeStruct((NC, n_pad, D), jnp.float32),
        mesh=mesh,
        scratch_types=[
            pltpu.VMEM_SHARED((n_pad, D), jnp.float32),
            [pltpu.VMEM((C_SZ,), jnp.int32)] * 3,
            [pltpu.VMEM((C_SZ,), jnp.int32)] * 3,
            [pltpu.VMEM((C_SZ, D), jnp.float32)] * 2,
            [pltpu.SemaphoreType.DMA] * 2,
            [pltpu.SemaphoreType.DMA] * 2,
            [pltpu.SemaphoreType.DMA] * 3,
        ],
    )
    def scatter_kernel(g, row1, col1, sp_out, acc, ridx, cidx, rows,
                       sem_g, sem_s, sem_i):
        cid = lax.axis_index("c")
        sid = lax.axis_index("s")
        wid = cid * NS + sid
        nb = sid * npt
        eb = wid * epw
        zbuf = rows[0]

        _flat_zero(zbuf, wr * D)
        for k in range(n_wr):
            pltpu.async_copy(zbuf, acc.at[pl.ds(nb + k * wr, wr)], sem_s[0])
        for k in range(n_wr):
            pltpu.make_async_copy(zbuf, acc.at[pl.ds(nb + k * wr, wr)],
                                  sem_s[0]).wait()
        plsc.subcore_barrier()

        def load_idx(j, b3):
            pltpu.async_copy(row1.at[pl.ds(eb + j * C_SZ, C_SZ)], ridx[b3],
                             sem_i[b3])
            pltpu.async_copy(col1.at[pl.ds(eb + j * C_SZ, C_SZ)], cidx[b3],
                             sem_i[b3])

        def wait_idx(j, b3):
            pltpu.make_async_copy(row1.at[pl.ds(eb + j * C_SZ, C_SZ)],
                                  ridx[b3], sem_i[b3]).wait()
            pltpu.make_async_copy(col1.at[pl.ds(eb + j * C_SZ, C_SZ)],
                                  cidx[b3], sem_i[b3]).wait()

        def wait_gather(b2, b3):
            pltpu.make_async_copy(g.at[cidx[b3]], rows[b2], sem_g[b2]).wait()

        def wait_scatter(b2, b3):
            pltpu.make_async_copy(rows[b2], acc.at[ridx[b3]],
                                  sem_s[b2]).wait()

        load_idx(0, 0)
        wait_idx(0, 0)
        pltpu.async_copy(g.at[cidx[0]], rows[0], sem_g[0])
        load_idx(1, 1)

        def step(j, jb):
            b2, b3 = jb % 2, jb % 3
            n2, n3 = (jb + 1) % 2, (jb + 1) % 3

            @pl.when(j + 1 < nch)
            def _():
                wait_idx(j + 1, n3)

                @pl.when(j >= 1)
                def _():
                    wait_scatter(n2, (jb + 2) % 3)
                pltpu.async_copy(g.at[cidx[n3]], rows[n2], sem_g[n2])

            wait_gather(b2, b3)
            pltpu.async_copy(rows[b2], acc.at[ridx[b3]], sem_s[b2], add=True)

            @pl.when(j + 2 < nch)
            def _():
                load_idx(j + 2, (jb + 2) % 3)

        def six(i6, _):
            for jb in range(6):
                step(i6 * 6 + jb, jb)
            return 0

        lax.fori_loop(0, nch // 6, six, 0)
        for j in range(nch - nch % 6, nch):
            step(j, j % 6)
        last = nch - 1
        wait_scatter((last - 1) % 2, (last - 1) % 3)
        wait_scatter(last % 2, last % 3)
        plsc.subcore_barrier()

        def rd(k, b):
            return (acc.at[pl.ds(nb + k * wr, wr)], rows[b], sem_g[b])

        def wo(k, b):
            return (rows[b], sp_out.at[cid, pl.ds(nb + k * wr, wr)], sem_s[b])

        pltpu.async_copy(*rd(0, 0))
        for k in range(n_wr):
            b = k % 2
            if k + 1 < n_wr:
                if k >= 1:
                    pltpu.make_async_copy(*wo(k - 1, 1 - b)).wait()
                pltpu.async_copy(*rd(k + 1, 1 - b))
            pltpu.make_async_copy(*rd(k, b)).wait()
            pltpu.async_copy(*wo(k, b))
        if n_wr >= 2:
            pltpu.make_async_copy(*wo(n_wr - 2, (n_wr - 2) % 2)).wait()
        pltpu.make_async_copy(*wo(n_wr - 1, (n_wr - 1) % 2)).wait()

    return scatter_kernel


def _scale_body(deg_ref, x_ref, g_ref, dinv_ref):
    d = deg_ref[0].astype(jnp.float32) + deg_ref[1].astype(jnp.float32)
    dinv = lax.rsqrt(1.0 + d)
    g_ref[...] = dinv * x_ref[...]
    dinv_ref[...] = dinv


def _mlp_body(relu, sp_ref, g_ref, dinv_ref, w_ref, b_ref, o_ref):
    dinv = dinv_ref[...]
    h1 = dinv * (sp_ref[0] + sp_ref[1] + g_ref[...])
    h = lax.dot_general(h1, w_ref[...], (((1,), (1,)), ((), ())),
                        preferred_element_type=jnp.float32) + b_ref[...]
    if relu:
        o_ref[...] = dinv * jnp.maximum(h, 0.0)
    else:
        o_ref[...] = h


def _tc_scale(deg_parts, x, nb):
    n = x.shape[0]
    return pl.pallas_call(
        _scale_body,
        grid=(n // nb,),
        in_specs=[
            pl.BlockSpec((NC, nb, D), lambda i: (0, i, 0)),
            pl.BlockSpec((nb, D), lambda i: (i, 0)),
        ],
        out_specs=[
            pl.BlockSpec((nb, D), lambda i: (i, 0)),
            pl.BlockSpec((nb, D), lambda i: (i, 0)),
        ],
        out_shape=[
            jax.ShapeDtypeStruct((n, D), jnp.float32),
            jax.ShapeDtypeStruct((n, D), jnp.float32),
        ],
    )(deg_parts, x)


def _tc_mlp(sp, g, dinvb, w, b2d, nb, relu):
    n = g.shape[0]
    return pl.pallas_call(
        functools.partial(_mlp_body, relu),
        grid=(n // nb,),
        in_specs=[
            pl.BlockSpec((NC, nb, D), lambda i: (0, i, 0)),
            pl.BlockSpec((nb, D), lambda i: (i, 0)),
            pl.BlockSpec((nb, D), lambda i: (i, 0)),
            pl.BlockSpec((D, D), lambda i: (0, 0)),
            pl.BlockSpec((1, D), lambda i: (0, 0)),
        ],
        out_specs=pl.BlockSpec((nb, D), lambda i: (i, 0)),
        out_shape=jax.ShapeDtypeStruct((n, D), jnp.float32),
    )(sp, g, dinvb, w, b2d)


@jax.jit
def kernel(x, edge_index, W1, b1, W2, b2):
    n = x.shape[0]
    e = edge_index.shape[1]
    n_pad = ((n + NW * 8 - 1) // (NW * 8)) * (NW * 8)
    nb = 1000

    e_pad = ((e + NW * C_SZ - 1) // (NW * C_SZ)) * (NW * C_SZ)
    pad = e_pad - e
    row1 = jnp.concatenate(
        [edge_index[0].astype(jnp.int32),
         n + jnp.arange(pad, dtype=jnp.int32) % (n_pad - n)])
    col1 = jnp.concatenate(
        [edge_index[1].astype(jnp.int32),
         jnp.arange(pad, dtype=jnp.int32) % n])

    deg_parts = _make_deg_kernel(n_pad, e_pad)(row1)
    g, dinvb = _tc_scale(deg_parts, x, nb)

    edge_scatter = _make_scatter_kernel(n_pad, e_pad)
    s1 = edge_scatter(g, row1, col1)
    g2 = _tc_mlp(s1, g, dinvb, W1, b1.reshape(1, D), nb, relu=True)
    s2 = edge_scatter(g2, row1, col1)
    return _tc_mlp(s2, g2, dinvb, W2, b2.reshape(1, D), nb, relu=False)
